# baseline (speedup 1.0000x reference)
import functools

import jax
import jax.numpy as jnp
from jax import lax
from jax.experimental import pallas as pl
from jax.experimental.pallas import tpu as pltpu
from jax.experimental.pallas import tpu_sc as plsc

SEQ = 200
D = 64
DP = 128
BATCH = 4096
NW = 32
VOCAB = 1000000
SEQ_PER_W = BATCH // NW
SEQ_PER_CHUNK = 4
TOK_PER_CHUNK = SEQ_PER_CHUNK * SEQ
CHUNKS = SEQ_PER_W // SEQ_PER_CHUNK
TOK_PER_W = SEQ_PER_W * SEQ
GSUB = 128


BLKW = 128
NBLK = (VOCAB - D) // BLKW


def _sc_repack(emb_t, tail_pack):
    mesh = plsc.VectorSubcoreMesh(core_axis_name="c", subcore_axis_name="s")

    @functools.partial(
        pl.kernel,
        mesh=mesh,
        out_type=jax.ShapeDtypeStruct((VOCAB // 2, 128), jnp.float32),
        scratch_types=[
            pltpu.VMEM((D, BLKW), jnp.float32),
            pltpu.VMEM((D, BLKW), jnp.float32),
            pltpu.VMEM((BLKW // 2, 128), jnp.float32),
            pltpu.SemaphoreType.DMA,
            pltpu.SemaphoreType.DMA,
        ],
        compiler_params=pltpu.CompilerParams(needs_layout_passes=False),
    )
    def k(embt_hbm, tail_hbm, out_hbm, blk_a, blk_b, stage_v, sem_a, sem_b):
        wid = lax.axis_index("s") * 2 + lax.axis_index("c")
        iota = lax.broadcasted_iota(jnp.int32, (16,), 0)
        rowidx = [16 * j + iota for j in range(D // 16)]

        def fetch(g, blk, sem):
            pltpu.async_copy(
                embt_hbm.at[:, pl.ds(g * BLKW, BLKW)], blk, sem)

        def drain(g, blk, sem):
            pltpu.make_async_copy(
                embt_hbm.at[:, pl.ds(g * BLKW, BLKW)], blk, sem).wait()

        def transpose(g, blk):
            def tok_body(k2, carry):
                krow = lax.broadcast(k2, (16,))
                for h in range(2):
                    col = lax.broadcast(2 * k2 + h, (16,))
                    for j in range(D // 16):
                        v = plsc.load_gather(blk, [rowidx[j], col])
                        plsc.store_scatter(
                            stage_v, [krow, 64 * h + 16 * j + iota], v)
                return carry

            lax.fori_loop(0, BLKW // 2, tok_body, 0)
            pltpu.sync_copy(stage_v, out_hbm.at[pl.ds(g * (BLKW // 2),
                                                      BLKW // 2)])

        def blk_id(b):
            return b * NW + wid

        nloop = (NBLK + NW - 1) // NW

        fetch(blk_id(0), blk_a, sem_a)

        def pair_body(i, carry):
            ga = blk_id(2 * i)
            gb = blk_id(2 * i + 1)

            @pl.when(gb < NBLK)
            def _():
                fetch(gb, blk_b, sem_b)

            @pl.when(ga < NBLK)
            def _():
                drain(ga, blk_a, sem_a)
                transpose(ga, blk_a)

            @pl.when(blk_id(2 * i + 2) < NBLK)
            def _():
                fetch(blk_id(2 * i + 2), blk_a, sem_a)

            @pl.when(gb < NBLK)
            def _():
                drain(gb, blk_b, sem_b)
                transpose(gb, blk_b)
            return carry

        lax.fori_loop(0, (nloop + 1) // 2, pair_body, 0)

        @pl.when(wid == 0)
        def _():
            pltpu.sync_copy(tail_hbm, stage_v.at[pl.ds(0, D // 2)])
            pltpu.sync_copy(stage_v.at[pl.ds(0, D // 2)],
                            out_hbm.at[pl.ds((VOCAB - D) // 2, D // 2)])

    return k(emb_t, tail_pack)


def _sc_embed(ids_flat, emb_pad, pos):
    mesh = plsc.VectorSubcoreMesh(core_axis_name="c", subcore_axis_name="s")

    @functools.partial(
        pl.kernel,
        mesh=mesh,
        out_type=jax.ShapeDtypeStruct((BATCH, SEQ, DP), jnp.float32),
        scratch_types=[
            pltpu.VMEM((TOK_PER_CHUNK,), jnp.int32),
            pltpu.VMEM((TOK_PER_CHUNK,), jnp.int32),
            pltpu.VMEM((SEQ_PER_CHUNK, SEQ, D), jnp.float32),
            pltpu.VMEM((SEQ_PER_CHUNK, SEQ, D), jnp.float32),
            pltpu.VMEM((SEQ, D), jnp.float32),
            pltpu.SemaphoreType.DMA,
            pltpu.SemaphoreType.DMA,
            pltpu.SemaphoreType.DMA,
            pltpu.SemaphoreType.DMA,
        ],
        compiler_params=pltpu.CompilerParams(use_tc_tiling_on_sc=False),
    )
    def k(ids_hbm, emb_hbm, pos_hbm, out_hbm,
          idx_a, idx_b, rows_a, rows_b, pos_v,
          gsem_a, gsem_b, osem_a, osem_b):
        wid = lax.axis_index("s") * 2 + lax.axis_index("c")
        base = wid * TOK_PER_W

        pltpu.sync_copy(pos_hbm.at[pl.ds(0, SEQ)], pos_v)

        def out_slice(c):
            return out_hbm.at[pl.ds(wid * SEQ_PER_W + c * SEQ_PER_CHUNK,
                                    SEQ_PER_CHUNK), :, pl.ds(0, D)]

        def fire_gathers(c, idx_v, rows_v, gsem):
            tok0 = base + c * TOK_PER_CHUNK
            pltpu.sync_copy(ids_hbm.at[pl.ds(tok0, TOK_PER_CHUNK)], idx_v)
            for s in range(SEQ_PER_CHUNK):
                off = 0
                while off < SEQ:
                    n = min(GSUB, SEQ - off)
                    pltpu.async_copy(
                        emb_hbm.at[idx_v.at[pl.ds(s * SEQ + off, n)]],
                        rows_v.at[s].at[pl.ds(off, n)],
                        gsem,
                    )
                    off += n

        def drain_gathers(idx_v, rows_v, gsem):
            for s in range(SEQ_PER_CHUNK):
                off = 0
                while off < SEQ:
                    n = min(GSUB, SEQ - off)
                    pltpu.make_async_copy(
                        emb_hbm.at[idx_v.at[pl.ds(s * SEQ + off, n)]],
                        rows_v.at[s].at[pl.ds(off, n)],
                        gsem,
                    ).wait()
                    off += n

        def add_pos(rows_v):
            def pos_body(p, carry2):
                for j in range(D // 16):
                    pv = pos_v[p, pl.ds(16 * j, 16)]
                    for s in range(SEQ_PER_CHUNK):
                        plsc.addupdate(rows_v.at[s, p, pl.ds(16 * j, 16)], pv)
                return carry2

            lax.fori_loop(0, SEQ, pos_body, 0)

        def wait_wb(c, rows_v, osem):
            pltpu.make_async_copy(rows_v, out_slice(c), osem).wait()

        fire_gathers(0, idx_a, rows_a, gsem_a)

        def pair_body(i, carry):
            ca = 2 * i

            @pl.when(i > 0)
            def _():
                wait_wb(ca - 1, rows_b, osem_b)

            fire_gathers(ca + 1, idx_b, rows_b, gsem_b)
            drain_gathers(idx_a, rows_a, gsem_a)
            add_pos(rows_a)
            pltpu.async_copy(rows_a, out_slice(ca), osem_a)

            @pl.when(i + 1 < CHUNKS // 2)
            def _():
                wait_wb(ca, rows_a, osem_a)
                fire_gathers(ca + 2, idx_a, rows_a, gsem_a)

            drain_gathers(idx_b, rows_b, gsem_b)
            add_pos(rows_b)
            pltpu.async_copy(rows_b, out_slice(ca + 1), osem_b)
            return carry

        lax.fori_loop(0, CHUNKS // 2, pair_body, 0)
        wait_wb(CHUNKS - 2, rows_a, osem_a)
        wait_wb(CHUNKS - 1, rows_b, osem_b)

    return k(ids_flat, emb_pad, pos)


def kernel(input_ids, embedding, position_embedding):
    ids_flat = input_ids.reshape(-1).astype(jnp.int32)
    tail_pack = embedding[VOCAB - D:].reshape(D // 2, 128)
    emb_lin = _sc_repack(embedding.T, tail_pack)
    out = _sc_embed(ids_flat, emb_lin.reshape(VOCAB, D), position_embedding)
    return out[:, :, :D]


# speedup vs baseline: 2.0407x; 2.0407x over previous
import functools

import jax
import jax.numpy as jnp
from jax import lax
from jax.experimental import pallas as pl
from jax.experimental.pallas import tpu as pltpu
from jax.experimental.pallas import tpu_sc as plsc

SEQ = 200
D = 64
DP = 128
BATCH = 4096
NW = 32
VOCAB = 1000000
SEQ_PER_W = BATCH // NW
SEQ_PER_CHUNK = 4
TOK_PER_CHUNK = SEQ_PER_CHUNK * SEQ
CHUNKS = SEQ_PER_W // SEQ_PER_CHUNK
TOK_PER_W = SEQ_PER_W * SEQ
GSUB = 128


def _sc_embed(ids_flat, emb_pad, pos):
    mesh = plsc.VectorSubcoreMesh(core_axis_name="c", subcore_axis_name="s")

    @functools.partial(
        pl.kernel,
        mesh=mesh,
        out_type=jax.ShapeDtypeStruct((BATCH, SEQ, DP), jnp.float32),
        scratch_types=[
            pltpu.VMEM((TOK_PER_CHUNK,), jnp.int32),
            pltpu.VMEM((TOK_PER_CHUNK,), jnp.int32),
            pltpu.VMEM((SEQ_PER_CHUNK, SEQ, D), jnp.float32),
            pltpu.VMEM((SEQ_PER_CHUNK, SEQ, D), jnp.float32),
            pltpu.VMEM((SEQ, D), jnp.float32),
            pltpu.SemaphoreType.DMA,
            pltpu.SemaphoreType.DMA,
            pltpu.SemaphoreType.DMA,
            pltpu.SemaphoreType.DMA,
        ],
        compiler_params=pltpu.CompilerParams(use_tc_tiling_on_sc=False),
    )
    def k(ids_hbm, emb_hbm, pos_hbm, out_hbm,
          idx_a, idx_b, rows_a, rows_b, pos_v,
          gsem_a, gsem_b, osem_a, osem_b):
        wid = lax.axis_index("s") * 2 + lax.axis_index("c")
        base = wid * TOK_PER_W

        pltpu.sync_copy(pos_hbm.at[pl.ds(0, SEQ)], pos_v)

        def out_slice(c):
            return out_hbm.at[pl.ds(wid * SEQ_PER_W + c * SEQ_PER_CHUNK,
                                    SEQ_PER_CHUNK), :, pl.ds(0, D)]

        def fire_gathers(c, idx_v, rows_v, gsem):
            tok0 = base + c * TOK_PER_CHUNK
            pltpu.sync_copy(ids_hbm.at[pl.ds(tok0, TOK_PER_CHUNK)], idx_v)
            for s in range(SEQ_PER_CHUNK):
                off = 0
                while off < SEQ:
                    n = min(GSUB, SEQ - off)
                    pltpu.async_copy(
                        emb_hbm.at[idx_v.at[pl.ds(s * SEQ + off, n)]],
                        rows_v.at[s].at[pl.ds(off, n)],
                        gsem,
                    )
                    off += n

        def drain_gathers(idx_v, rows_v, gsem):
            for s in range(SEQ_PER_CHUNK):
                off = 0
                while off < SEQ:
                    n = min(GSUB, SEQ - off)
                    pltpu.make_async_copy(
                        emb_hbm.at[idx_v.at[pl.ds(s * SEQ + off, n)]],
                        rows_v.at[s].at[pl.ds(off, n)],
                        gsem,
                    ).wait()
                    off += n

        def add_pos(rows_v):
            def pos_body(p, carry2):
                for j in range(D // 16):
                    pv = pos_v[p, pl.ds(16 * j, 16)]
                    for s in range(SEQ_PER_CHUNK):
                        plsc.addupdate(rows_v.at[s, p, pl.ds(16 * j, 16)], pv)
                return carry2

            lax.fori_loop(0, SEQ, pos_body, 0)

        def wait_wb(c, rows_v, osem):
            pltpu.make_async_copy(rows_v, out_slice(c), osem).wait()

        fire_gathers(0, idx_a, rows_a, gsem_a)

        def pair_body(i, carry):
            ca = 2 * i

            @pl.when(i > 0)
            def _():
                wait_wb(ca - 1, rows_b, osem_b)

            fire_gathers(ca + 1, idx_b, rows_b, gsem_b)
            drain_gathers(idx_a, rows_a, gsem_a)
            add_pos(rows_a)
            pltpu.async_copy(rows_a, out_slice(ca), osem_a)

            @pl.when(i + 1 < CHUNKS // 2)
            def _():
                wait_wb(ca, rows_a, osem_a)
                fire_gathers(ca + 2, idx_a, rows_a, gsem_a)

            drain_gathers(idx_b, rows_b, gsem_b)
            add_pos(rows_b)
            pltpu.async_copy(rows_b, out_slice(ca + 1), osem_b)
            return carry

        lax.fori_loop(0, CHUNKS // 2, pair_body, 0)
        wait_wb(CHUNKS - 2, rows_a, osem_a)
        wait_wb(CHUNKS - 1, rows_b, osem_b)

    return k(ids_flat, emb_pad, pos)


def kernel(input_ids, embedding, position_embedding):
    ids_flat = input_ids.reshape(-1).astype(jnp.int32)
    out = _sc_embed(ids_flat, embedding, position_embedding)
    return out[:, :, :D]
